# baseline (speedup 1.0000x reference)
import jax
import jax.numpy as jnp
from jax.experimental import pallas as pl
from jax.experimental.pallas import tpu as pltpu

B, C, H, W = 2, 96, 512, 512
HB = 64
NH = H // HB
WP = W + 8


def _coef_body(w_ref, psi_ref, coef_ref):
    coef_ref[...] = jax.lax.dot(
        w_ref[...], psi_ref[...], preferred_element_type=jnp.float32
    )


def _conv_body(coef_ref, bias_ref, x_ref, o_ref, xp_ref):
    c = pl.program_id(1)
    h = pl.program_id(2)

    xp_ref[...] = jnp.zeros((HB + 16, WP), jnp.float32)

    @pl.when(h == 0)
    def _():
        xp_ref[8:HB + 16, 2:W + 2] = x_ref[0, 0, 0:HB + 8, :]

    @pl.when(h == NH - 1)
    def _():
        xp_ref[0:HB + 8, 2:W + 2] = x_ref[0, 0, H - HB - 8:H, :]

    @pl.when(jnp.logical_and(h > 0, h < NH - 1))
    def _():
        start = pl.multiple_of(h * HB - 8, 8)
        xp_ref[0:HB + 16, 2:W + 2] = x_ref[0, 0, pl.ds(start, HB + 16), :]

    acc = jnp.full((HB, W), bias_ref[c], jnp.float32)
    for dy in range(5):
        for dx in range(5):
            coeff = coef_ref[c, 5 * dy + dx]
            acc = acc + coeff * xp_ref[6 + dy:6 + dy + HB, dx:dx + W]
    o_ref[0, 0] = acc


def kernel(x, weight, bias, psi_loc):
    w2 = weight.reshape(C, -1)[:, -3:]
    psi25 = psi_loc[:, 1:6, 1:6].reshape(3, 25)

    coef = pl.pallas_call(
        _coef_body,
        out_shape=jax.ShapeDtypeStruct((C, 25), jnp.float32),
    )(w2, psi25)

    out = pl.pallas_call(
        _conv_body,
        grid=(B, C, NH),
        in_specs=[
            pl.BlockSpec(memory_space=pltpu.SMEM),
            pl.BlockSpec(memory_space=pltpu.SMEM),
            pl.BlockSpec((1, 1, H, W), lambda b, c, h: (b, c, 0, 0)),
        ],
        out_specs=pl.BlockSpec((1, 1, HB, W), lambda b, c, h: (b, c, h, 0)),
        out_shape=jax.ShapeDtypeStruct((B, C, H, W), jnp.float32),
        scratch_shapes=[pltpu.VMEM((HB + 16, WP), jnp.float32)],
    )(coef, bias, x)
    return out


# speedup vs baseline: 2.5391x; 2.5391x over previous
import jax
import jax.numpy as jnp
from jax.experimental import pallas as pl
from jax.experimental.pallas import tpu as pltpu

B, C, H, W = 2, 96, 512, 512
HB = 128
NH = H // HB
WP = W + 8


def _coef_body(w_ref, psi_ref, coef_ref):
    coef_ref[...] = jax.lax.dot(
        w_ref[...], psi_ref[...], preferred_element_type=jnp.float32
    )


def _conv_body(coef_ref, bias_ref, x_ref, o_ref, xp_ref):
    c = pl.program_id(1)
    h = pl.program_id(2)

    xp_ref[...] = jnp.zeros((HB + 16, WP), jnp.float32)

    @pl.when(h == 0)
    def _():
        xp_ref[8:HB + 16, 2:W + 2] = x_ref[0, 0, 0:HB + 8, :]

    @pl.when(h == NH - 1)
    def _():
        xp_ref[0:HB + 8, 2:W + 2] = x_ref[0, 0, H - HB - 8:H, :]

    @pl.when(jnp.logical_and(h > 0, h < NH - 1))
    def _():
        start = pl.multiple_of(h * HB - 8, 8)
        xp_ref[0:HB + 16, 2:W + 2] = x_ref[0, 0, pl.ds(start, HB + 16), :]

    rows = [xp_ref[6 + dy:6 + dy + HB, :] for dy in range(5)]
    s04 = rows[0] + rows[4]
    s13 = rows[1] + rows[3]
    s2 = rows[2]
    tmps = []
    for dx in range(3):
        tmp = (coef_ref[c, dx] * s04
               + coef_ref[c, 5 + dx] * s13
               + coef_ref[c, 10 + dx] * s2)
        tmps.append(tmp)

    acc = jnp.full((HB, W), bias_ref[c], jnp.float32)
    acc = acc + tmps[0][:, 0:W] + tmps[1][:, 1:1 + W] + tmps[2][:, 2:2 + W]
    acc = acc + tmps[1][:, 3:3 + W] + tmps[0][:, 4:4 + W]
    o_ref[0, 0] = acc


def kernel(x, weight, bias, psi_loc):
    w2 = weight.reshape(C, -1)[:, -3:]
    psi25 = psi_loc[:, 1:6, 1:6].reshape(3, 25)

    coef = pl.pallas_call(
        _coef_body,
        out_shape=jax.ShapeDtypeStruct((C, 25), jnp.float32),
    )(w2, psi25)

    out = pl.pallas_call(
        _conv_body,
        grid=(B, C, NH),
        in_specs=[
            pl.BlockSpec(memory_space=pltpu.SMEM),
            pl.BlockSpec(memory_space=pltpu.SMEM),
            pl.BlockSpec((1, 1, H, W), lambda b, c, h: (b, c, 0, 0)),
        ],
        out_specs=pl.BlockSpec((1, 1, HB, W), lambda b, c, h: (b, c, h, 0)),
        out_shape=jax.ShapeDtypeStruct((B, C, H, W), jnp.float32),
        scratch_shapes=[pltpu.VMEM((HB + 16, WP), jnp.float32)],
    )(coef, bias, x)
    return out
